# Initial kernel scaffold; baseline (speedup 1.0000x reference)
import jax
import jax.numpy as jnp
from jax.experimental import pallas as pl


def kernel(x, indices):
    raise NotImplementedError("write your pallas kernel here")


# speedup vs baseline: 2.4084x; 2.4084x over previous
import functools

import jax
import jax.numpy as jnp
from jax.experimental import pallas as pl
from jax.experimental.pallas import tpu as pltpu


def _roll_kernel(idx_ref, x_ref, o_ref, *, cb, c_total):
    q = c_total // 4
    c0 = pl.program_id(1) * cb
    chan = c0 + jax.lax.broadcasted_iota(jnp.int32, (cb, c_total), 0)
    j = jax.lax.broadcasted_iota(jnp.int32, (cb, c_total), 1)
    idx = idx_ref[0][None, :]
    labels = jnp.sum(jnp.where(idx == chan, j // q, 0), axis=1)
    lab = labels.reshape(cb, 1, 1)

    x = x_ref[0]
    h, w = x.shape[1], x.shape[2]
    r_h_p = pltpu.roll(x, 1, 1)
    r_h_m = pltpu.roll(x, h - 1, 1)
    r_w_p = pltpu.roll(x, 1, 2)
    r_w_m = pltpu.roll(x, w - 1, 2)
    out = jnp.where(
        lab < 2,
        jnp.where(lab == 0, r_h_p, r_h_m),
        jnp.where(lab == 2, r_w_p, r_w_m),
    )
    o_ref[0] = out


@jax.jit
def kernel(x, indices):
    b, c, h, w = x.shape
    cb = 16
    idx = indices.astype(jnp.int32).reshape(1, c)
    grid = (b, c // cb)
    return pl.pallas_call(
        functools.partial(_roll_kernel, cb=cb, c_total=c),
        grid=grid,
        in_specs=[
            pl.BlockSpec((1, c), lambda i, k: (0, 0)),
            pl.BlockSpec((1, cb, h, w), lambda i, k: (i, k, 0, 0)),
        ],
        out_specs=pl.BlockSpec((1, cb, h, w), lambda i, k: (i, k, 0, 0)),
        out_shape=jax.ShapeDtypeStruct((b, c, h, w), x.dtype),
    )(idx, x)


# speedup vs baseline: 2.5474x; 1.0577x over previous
import functools

import jax
import jax.numpy as jnp
from jax.experimental import pallas as pl
from jax.experimental.pallas import tpu as pltpu


def _roll_kernel(lab_ref, x_ref, o_ref, *, cb):
    c0 = pl.program_id(1) * cb
    h, w = x_ref.shape[2], x_ref.shape[3]

    def body(i, carry):
        lab = lab_ref[c0 + i]
        x = x_ref[0, i]

        def roll_h_p():
            o_ref[0, i] = pltpu.roll(x, 1, 0)

        def roll_h_m():
            o_ref[0, i] = pltpu.roll(x, h - 1, 0)

        def roll_w_p():
            o_ref[0, i] = pltpu.roll(x, 1, 1)

        def roll_w_m():
            o_ref[0, i] = pltpu.roll(x, w - 1, 1)

        jax.lax.switch(lab, [roll_h_p, roll_h_m, roll_w_p, roll_w_m])
        return carry

    jax.lax.fori_loop(0, cb, body, 0)


@jax.jit
def kernel(x, indices):
    b, c, h, w = x.shape
    q = c // 4
    cb = 16
    idx = indices.astype(jnp.int32)
    labels = jnp.zeros((c,), jnp.int32).at[idx].set(jnp.arange(c, dtype=jnp.int32) // q)
    grid_spec = pltpu.PrefetchScalarGridSpec(
        num_scalar_prefetch=1,
        grid=(b, c // cb),
        in_specs=[pl.BlockSpec((1, cb, h, w), lambda i, k, lab: (i, k, 0, 0))],
        out_specs=pl.BlockSpec((1, cb, h, w), lambda i, k, lab: (i, k, 0, 0)),
    )
    return pl.pallas_call(
        functools.partial(_roll_kernel, cb=cb),
        grid_spec=grid_spec,
        out_shape=jax.ShapeDtypeStruct((b, c, h, w), x.dtype),
    )(labels, x)


# speedup vs baseline: 2.5622x; 1.0058x over previous
import functools

import jax
import jax.numpy as jnp
from jax.experimental import pallas as pl
from jax.experimental.pallas import tpu as pltpu


def _roll_kernel(lab_ref, x_ref, o_ref, *, cb):
    c0 = pl.program_id(1) * cb
    h, w = x_ref.shape[2], x_ref.shape[3]

    for i in range(cb):
        lab = lab_ref[c0 + i]
        x = x_ref[0, i]

        def roll_h_p(x=x, i=i):
            o_ref[0, i] = pltpu.roll(x, 1, 0)

        def roll_h_m(x=x, i=i):
            o_ref[0, i] = pltpu.roll(x, h - 1, 0)

        def roll_w_p(x=x, i=i):
            o_ref[0, i] = pltpu.roll(x, 1, 1)

        def roll_w_m(x=x, i=i):
            o_ref[0, i] = pltpu.roll(x, w - 1, 1)

        jax.lax.switch(lab, [roll_h_p, roll_h_m, roll_w_p, roll_w_m])


@jax.jit
def kernel(x, indices):
    b, c, h, w = x.shape
    q = c // 4
    cb = 16
    idx = indices.astype(jnp.int32)
    labels = jnp.zeros((c,), jnp.int32).at[idx].set(jnp.arange(c, dtype=jnp.int32) // q)
    grid_spec = pltpu.PrefetchScalarGridSpec(
        num_scalar_prefetch=1,
        grid=(b, c // cb),
        in_specs=[pl.BlockSpec((1, cb, h, w), lambda i, k, lab: (i, k, 0, 0))],
        out_specs=pl.BlockSpec((1, cb, h, w), lambda i, k, lab: (i, k, 0, 0)),
    )
    return pl.pallas_call(
        functools.partial(_roll_kernel, cb=cb),
        grid_spec=grid_spec,
        out_shape=jax.ShapeDtypeStruct((b, c, h, w), x.dtype),
    )(labels, x)
